# baseline (speedup 1.0000x reference)
import functools

import jax
import jax.numpy as jnp
import numpy as np
from jax import lax
from jax.experimental import pallas as pl
from jax.experimental.pallas import tpu as pltpu
from jax.experimental.pallas import tpu_sc as plsc

_MARGIN = 0.2
_B, _QL, _AL, _D = 4096, 20, 50, 64
_NC, _NS = 2, 16
_NW = _NC * _NS
_R = _B // _NW
_CH = 128
_TOK = _R * (_QL + 2 * _AL)
_NCHUNK = _TOK // _CH

_dest = np.concatenate([
    np.arange(_R * _QL, dtype=np.int32) // _QL,
    _R + np.arange(_R * _AL, dtype=np.int32) // _AL,
    2 * _R + np.arange(_R * _AL, dtype=np.int32) // _AL,
]).reshape(_NCHUNK, _CH)
_dest_per_subcore = (
    np.arange(_NS, dtype=np.int32)[:, None, None] * (3 * _R) + _dest[None]
)


def _make_pool_kernel():
    mesh = plsc.VectorSubcoreMesh(core_axis_name="c", subcore_axis_name="s")
    f32 = jnp.float32

    @functools.partial(
        pl.kernel,
        out_type=[jax.ShapeDtypeStruct((_B, _D), f32)] * 3,
        mesh=mesh,
        scratch_types=[
            pltpu.VMEM((_NCHUNK, _CH), jnp.int32),
            pltpu.VMEM((_NCHUNK, _CH), jnp.int32),
            pltpu.VMEM((_CH, _D), f32),
            pltpu.VMEM((_CH, _D), f32),
            pltpu.VMEM_SHARED((_NS * 3 * _R, _D), f32),
            pltpu.SemaphoreType.DMA,
            pltpu.SemaphoreType.DMA,
        ],
        compiler_params=pltpu.CompilerParams(use_tc_tiling_on_sc=False),
    )
    def pool(tok_hbm, dest_hbm, zeros_hbm, table_hbm, out_q, out_g, out_b,
             tok_v, dest_v, buf_a, buf_b, acc, sem_a, sem_b):
        c = lax.axis_index("c")
        s = lax.axis_index("s")
        w = s * _NC + c

        pltpu.sync_copy(tok_hbm.at[w], tok_v)
        pltpu.sync_copy(dest_hbm.at[s], dest_v)
        pltpu.sync_copy(zeros_hbm, acc.at[pl.ds(s * 3 * _R, 3 * _R)])

        pltpu.async_copy(table_hbm.at[tok_v.at[0]], buf_a, sem_a)

        def body(i, carry):
            j = 2 * i
            pltpu.async_copy(table_hbm.at[tok_v.at[j + 1]], buf_b, sem_b)
            pltpu.make_async_copy(table_hbm.at[tok_v.at[j]], buf_a, sem_a).wait()
            pltpu.sync_copy(buf_a, acc.at[dest_v.at[j]], add=True)

            @pl.when(j + 2 < _NCHUNK)
            def _():
                pltpu.async_copy(table_hbm.at[tok_v.at[j + 2]], buf_a, sem_a)

            pltpu.make_async_copy(table_hbm.at[tok_v.at[j + 1]], buf_b, sem_b).wait()
            pltpu.sync_copy(buf_b, acc.at[dest_v.at[j + 1]], add=True)
            return carry

        lax.fori_loop(0, _NCHUNK // 2, body, 0)

        base = w * _R
        sbase = s * 3 * _R
        pltpu.sync_copy(acc.at[pl.ds(sbase, _R)], out_q.at[pl.ds(base, _R)])
        pltpu.sync_copy(acc.at[pl.ds(sbase + _R, _R)], out_g.at[pl.ds(base, _R)])
        pltpu.sync_copy(acc.at[pl.ds(sbase + 2 * _R, _R)], out_b.at[pl.ds(base, _R)])

    return pool


_pool_kernel = _make_pool_kernel()


_V = 1000000
_MAIN = 999936


def _eye128():
    return (jax.lax.broadcasted_iota(jnp.int32, (128, 128), 0) ==
            jax.lax.broadcasted_iota(jnp.int32, (128, 128), 1)).astype(jnp.float32)


def _make_transpose_body(groups, n_aliased=0):
    def body(*refs):
        a_ref, out_ref = refs[n_aliased], refs[n_aliased + 1]
        eye = _eye128()
        for g in range(groups):
            ab = jnp.concatenate(
                [a_ref[:, 256 * g: 256 * g + 128],
                 a_ref[:, 256 * g + 128: 256 * g + 256]], axis=0)
            out_ref[128 * g: 128 * (g + 1), :] = jax.lax.dot_general(
                ab, eye,
                dimension_numbers=(((0,), (0,)), ((), ())),
                preferred_element_type=jnp.float32,
                precision=jax.lax.Precision.HIGHEST,
            )
    return body


def _tail_body(_, a_ref, out_ref):
    t = a_ref[...].T
    out_ref[...] = jnp.concatenate([t[0:32], t[32:64]], axis=1)


def _linearize_table(embeddings):
    emb_t = embeddings.T
    f32 = jnp.float32
    lin = pl.pallas_call(
        _make_transpose_body(8),
        grid=(_V // 2048,),
        in_specs=[pl.BlockSpec((_D, 2048), lambda i: (0, i))],
        out_specs=pl.BlockSpec((1024, 128), lambda i: (i, 0)),
        out_shape=jax.ShapeDtypeStruct((_V // 2, 128), f32),
    )(emb_t)
    lin = pl.pallas_call(
        _make_transpose_body(2, n_aliased=1),
        grid=(1,),
        in_specs=[
            pl.BlockSpec(memory_space=pl.ANY),
            pl.BlockSpec((_D, 512), lambda i: (0, 1952)),
        ],
        out_specs=pl.BlockSpec((256, 128), lambda i: (1952, 0)),
        out_shape=jax.ShapeDtypeStruct((_V // 2, 128), f32),
        input_output_aliases={0: 0},
    )(lin, emb_t)
    tail = jax.lax.slice(emb_t, (0, _MAIN), (_D, _V))
    lin = pl.pallas_call(
        _tail_body,
        grid=(1,),
        in_specs=[
            pl.BlockSpec(memory_space=pl.ANY),
            pl.BlockSpec((_D, 64), lambda i: (0, 0)),
        ],
        out_specs=pl.BlockSpec((32, 128), lambda i: (_MAIN // 64, 0)),
        out_shape=jax.ShapeDtypeStruct((_V // 2, 128), f32),
        input_output_aliases={0: 0},
    )(lin, tail)
    return lin.reshape(_V, _D)


def _sigma(r):
    p = r - _MAIN
    return jnp.where(
        r < _MAIN,
        2 * (128 * (r // 256) + r % 128) + (r % 256) // 128,
        _MAIN + 2 * (p % 32) + p // 32,
    )


def _sim_body(q_ref, g_ref, b_ref, li_ref, pred_ref, loss_ref):
    q = q_ref[...]
    g = g_ref[...]
    b = b_ref[...]
    qg = jnp.sum(q * g, axis=1, keepdims=True)
    qb = jnp.sum(q * b, axis=1, keepdims=True)
    qq = jnp.sum(q * q, axis=1, keepdims=True)
    gg = jnp.sum(g * g, axis=1, keepdims=True)
    bb = jnp.sum(b * b, axis=1, keepdims=True)
    rq = lax.rsqrt(qq)
    sim_g = qg * rq * lax.rsqrt(gg)
    sim_b = qb * rq * lax.rsqrt(bb)
    li = jnp.maximum(0.0, _MARGIN - sim_g + sim_b)
    li_ref[...] = li
    pred_ref[...] = sim_g
    loss_ref[...] = jnp.sum(li, axis=0, keepdims=True) * (1.0 / _B)


def kernel(input_question, input_answer_good, input_answer_bad, embeddings):
    q = input_question.reshape(_NW, _R * _QL)
    g = input_answer_good.reshape(_NW, _R * _AL)
    b = input_answer_bad.reshape(_NW, _R * _AL)
    tok = jnp.concatenate([q, g, b], axis=1).reshape(_NW, _NCHUNK, _CH)
    tok = _sigma(tok)
    dest = jnp.asarray(_dest_per_subcore)
    zeros = jnp.zeros((3 * _R, _D), jnp.float32)

    table_lin = _linearize_table(embeddings)
    sum_q, sum_g, sum_b = _pool_kernel(tok, dest, zeros, table_lin)

    f32 = jnp.float32
    li, pred, loss = pl.pallas_call(
        _sim_body,
        out_shape=[
            jax.ShapeDtypeStruct((_B, 1), f32),
            jax.ShapeDtypeStruct((_B, 1), f32),
            jax.ShapeDtypeStruct((1, 1), f32),
        ],
    )(sum_q, sum_g, sum_b)

    return (loss[0, 0], li[:, 0], pred[:, 0])


# speedup vs baseline: 1.4779x; 1.4779x over previous
import functools

import jax
import jax.numpy as jnp
import numpy as np
from jax import lax
from jax.experimental import pallas as pl
from jax.experimental.pallas import tpu as pltpu
from jax.experimental.pallas import tpu_sc as plsc

_MARGIN = 0.2
_B, _QL, _AL, _D = 4096, 20, 50, 64
_NC, _NS = 2, 16
_NW = _NC * _NS
_R = _B // _NW
_CH = 128
_TOK = _R * (_QL + 2 * _AL)
_NCHUNK = _TOK // _CH
_NB = 4

_dest = np.concatenate([
    np.arange(_R * _QL, dtype=np.int32) // _QL,
    _R + np.arange(_R * _AL, dtype=np.int32) // _AL,
    2 * _R + np.arange(_R * _AL, dtype=np.int32) // _AL,
]).reshape(_NCHUNK, _CH)
_dest_per_subcore = (
    np.arange(_NS, dtype=np.int32)[:, None, None] * (3 * _R) + _dest[None]
)


def _make_pool_kernel():
    mesh = plsc.VectorSubcoreMesh(core_axis_name="c", subcore_axis_name="s")
    f32 = jnp.float32

    @functools.partial(
        pl.kernel,
        out_type=[jax.ShapeDtypeStruct((_B, _D), f32)] * 3,
        mesh=mesh,
        scratch_types=[
            pltpu.VMEM((_NCHUNK, _CH), jnp.int32),
            pltpu.VMEM((_NCHUNK, _CH), jnp.int32),
            [pltpu.VMEM((_CH, _D), f32)] * _NB,
            pltpu.VMEM_SHARED((_NS * 3 * _R, _D), f32),
            [pltpu.SemaphoreType.DMA] * _NB,
            [pltpu.SemaphoreType.DMA] * _NB,
        ],
        compiler_params=pltpu.CompilerParams(use_tc_tiling_on_sc=False),
    )
    def pool(tok_hbm, dest_hbm, zeros_hbm, table_hbm, out_q, out_g, out_b,
             tok_v, dest_v, bufs, acc, gsems, ssems):
        c = lax.axis_index("c")
        s = lax.axis_index("s")
        w = s * _NC + c

        pltpu.sync_copy(tok_hbm.at[w], tok_v)
        pltpu.sync_copy(dest_hbm.at[s], dest_v)
        pltpu.sync_copy(zeros_hbm, acc.at[pl.ds(s * 3 * _R, 3 * _R)])

        for b in range(_NB):
            pltpu.async_copy(table_hbm.at[tok_v.at[b]], bufs[b], gsems[b])

        def body(i, carry):
            for b in range(_NB):
                j = _NB * i + b
                pltpu.make_async_copy(
                    table_hbm.at[tok_v.at[j]], bufs[b], gsems[b]).wait()
                pltpu.async_copy(
                    bufs[b], acc.at[dest_v.at[j]], ssems[b], add=True)
            for b in range(_NB):
                j = _NB * i + b
                pltpu.make_async_copy(
                    bufs[b], acc.at[dest_v.at[j]], ssems[b]).wait()
                pltpu.async_copy(
                    table_hbm.at[tok_v.at[j + _NB]], bufs[b], gsems[b])
            return carry

        lax.fori_loop(0, _NCHUNK // _NB - 1, body, 0)

        for b in range(_NB):
            j = _NCHUNK - _NB + b
            pltpu.make_async_copy(
                table_hbm.at[tok_v.at[j]], bufs[b], gsems[b]).wait()
            pltpu.async_copy(bufs[b], acc.at[dest_v.at[j]], ssems[b], add=True)
        for b in range(_NB):
            j = _NCHUNK - _NB + b
            pltpu.make_async_copy(
                bufs[b], acc.at[dest_v.at[j]], ssems[b]).wait()

        base = w * _R
        sbase = s * 3 * _R
        pltpu.sync_copy(acc.at[pl.ds(sbase, _R)], out_q.at[pl.ds(base, _R)])
        pltpu.sync_copy(acc.at[pl.ds(sbase + _R, _R)], out_g.at[pl.ds(base, _R)])
        pltpu.sync_copy(acc.at[pl.ds(sbase + 2 * _R, _R)], out_b.at[pl.ds(base, _R)])

    return pool


_pool_kernel = _make_pool_kernel()


_V = 1000000
_MAIN = 999936


def _eye128():
    return (jax.lax.broadcasted_iota(jnp.int32, (128, 128), 0) ==
            jax.lax.broadcasted_iota(jnp.int32, (128, 128), 1)).astype(jnp.float32)


def _make_transpose_body(groups, n_aliased=0):
    def body(*refs):
        a_ref, out_ref = refs[n_aliased], refs[n_aliased + 1]
        eye = _eye128()
        for g in range(groups):
            ab = jnp.concatenate(
                [a_ref[:, 256 * g: 256 * g + 128],
                 a_ref[:, 256 * g + 128: 256 * g + 256]], axis=0)
            hi = jax.lax.bitcast_convert_type(
                jax.lax.bitcast_convert_type(ab, jnp.int32) & (-65536),
                jnp.float32)
            lo = ab - hi
            dims = (((0,), (0,)), ((), ()))
            out_ref[128 * g: 128 * (g + 1), :] = (
                jax.lax.dot_general(hi, eye, dims,
                                    preferred_element_type=jnp.float32)
                + jax.lax.dot_general(lo, eye, dims,
                                      preferred_element_type=jnp.float32)
            )
    return body


def _tail_body(_, a_ref, out_ref):
    t = a_ref[...].T
    out_ref[...] = jnp.concatenate([t[0:32], t[32:64]], axis=1)


def _linearize_table(embeddings):
    emb_t = embeddings.T
    f32 = jnp.float32
    lin = pl.pallas_call(
        _make_transpose_body(16),
        grid=(_V // 4096,),
        in_specs=[pl.BlockSpec((_D, 4096), lambda i: (0, i))],
        out_specs=pl.BlockSpec((2048, 128), lambda i: (i, 0)),
        out_shape=jax.ShapeDtypeStruct((_V // 2, 128), f32),
    )(emb_t)
    lin = pl.pallas_call(
        _make_transpose_body(2, n_aliased=1),
        grid=(1,),
        in_specs=[
            pl.BlockSpec(memory_space=pl.ANY),
            pl.BlockSpec((_D, 512), lambda i: (0, 1952)),
        ],
        out_specs=pl.BlockSpec((256, 128), lambda i: (1952, 0)),
        out_shape=jax.ShapeDtypeStruct((_V // 2, 128), f32),
        input_output_aliases={0: 0},
    )(lin, emb_t)
    tail = jax.lax.slice(emb_t, (0, _MAIN), (_D, _V))
    lin = pl.pallas_call(
        _tail_body,
        grid=(1,),
        in_specs=[
            pl.BlockSpec(memory_space=pl.ANY),
            pl.BlockSpec((_D, 64), lambda i: (0, 0)),
        ],
        out_specs=pl.BlockSpec((32, 128), lambda i: (_MAIN // 64, 0)),
        out_shape=jax.ShapeDtypeStruct((_V // 2, 128), f32),
        input_output_aliases={0: 0},
    )(lin, tail)
    return lin.reshape(_V, _D)


def _sigma(r):
    p = r - _MAIN
    return jnp.where(
        r < _MAIN,
        2 * (128 * (r // 256) + r % 128) + (r % 256) // 128,
        _MAIN + 2 * (p % 32) + p // 32,
    )


def _sim_body(q_ref, g_ref, b_ref, li_ref, pred_ref, loss_ref):
    q = q_ref[...]
    g = g_ref[...]
    b = b_ref[...]
    qg = jnp.sum(q * g, axis=1, keepdims=True)
    qb = jnp.sum(q * b, axis=1, keepdims=True)
    qq = jnp.sum(q * q, axis=1, keepdims=True)
    gg = jnp.sum(g * g, axis=1, keepdims=True)
    bb = jnp.sum(b * b, axis=1, keepdims=True)
    rq = lax.rsqrt(qq)
    sim_g = qg * rq * lax.rsqrt(gg)
    sim_b = qb * rq * lax.rsqrt(bb)
    li = jnp.maximum(0.0, _MARGIN - sim_g + sim_b)
    li_ref[...] = li
    pred_ref[...] = sim_g
    loss_ref[...] = jnp.sum(li, axis=0, keepdims=True) * (1.0 / _B)


def kernel(input_question, input_answer_good, input_answer_bad, embeddings):
    q = input_question.reshape(_NW, _R * _QL)
    g = input_answer_good.reshape(_NW, _R * _AL)
    b = input_answer_bad.reshape(_NW, _R * _AL)
    tok = jnp.concatenate([q, g, b], axis=1).reshape(_NW, _NCHUNK, _CH)
    tok = _sigma(tok)
    dest = jnp.asarray(_dest_per_subcore)
    zeros = jnp.zeros((3 * _R, _D), jnp.float32)

    table_lin = _linearize_table(embeddings)
    sum_q, sum_g, sum_b = _pool_kernel(tok, dest, zeros, table_lin)

    f32 = jnp.float32
    li, pred, loss = pl.pallas_call(
        _sim_body,
        out_shape=[
            jax.ShapeDtypeStruct((_B, 1), f32),
            jax.ShapeDtypeStruct((_B, 1), f32),
            jax.ShapeDtypeStruct((1, 1), f32),
        ],
    )(sum_q, sum_g, sum_b)

    return (loss[0, 0], li[:, 0], pred[:, 0])


# speedup vs baseline: 1.8399x; 1.2449x over previous
import functools

import jax
import jax.numpy as jnp
import numpy as np
from jax import lax
from jax.experimental import pallas as pl
from jax.experimental.pallas import tpu as pltpu
from jax.experimental.pallas import tpu_sc as plsc

_MARGIN = 0.2
_B, _QL, _AL, _D = 4096, 20, 50, 64
_NC, _NS = 2, 16
_NW = _NC * _NS
_R = _B // _NW
_CH = 128
_TOK = _R * (_QL + 2 * _AL)
_NCHUNK = _TOK // _CH
_NB = 8

_dest = np.concatenate([
    np.arange(_R * _QL, dtype=np.int32) // _QL,
    _R + np.arange(_R * _AL, dtype=np.int32) // _AL,
    2 * _R + np.arange(_R * _AL, dtype=np.int32) // _AL,
]).reshape(_NCHUNK, _CH)
_dest_per_subcore = (
    np.arange(_NS, dtype=np.int32)[:, None, None] * (3 * _R) + _dest[None]
)


def _make_pool_kernel():
    mesh = plsc.VectorSubcoreMesh(core_axis_name="c", subcore_axis_name="s")
    f32 = jnp.float32

    @functools.partial(
        pl.kernel,
        out_type=[jax.ShapeDtypeStruct((_B, _D), f32)] * 3,
        mesh=mesh,
        scratch_types=[
            pltpu.VMEM((_NCHUNK, _CH), jnp.int32),
            pltpu.VMEM((_NCHUNK, _CH), jnp.int32),
            [pltpu.VMEM((_CH, _D), f32)] * _NB,
            pltpu.VMEM_SHARED((_NS * 3 * _R, _D), f32),
            [pltpu.SemaphoreType.DMA] * _NB,
            [pltpu.SemaphoreType.DMA] * _NB,
        ],
        compiler_params=pltpu.CompilerParams(use_tc_tiling_on_sc=False),
    )
    def pool(tok_hbm, dest_hbm, zeros_hbm, table_hbm, out_q, out_g, out_b,
             tok_v, dest_v, bufs, acc, gsems, ssems):
        c = lax.axis_index("c")
        s = lax.axis_index("s")
        w = s * _NC + c

        pltpu.sync_copy(tok_hbm.at[w], tok_v)
        pltpu.sync_copy(dest_hbm.at[s], dest_v)
        pltpu.sync_copy(zeros_hbm, acc.at[pl.ds(s * 3 * _R, 3 * _R)])

        for b in range(_NB):
            pltpu.async_copy(table_hbm.at[tok_v.at[b]], bufs[b], gsems[b])

        def body(i, carry):
            for b in range(_NB):
                j = _NB * i + b
                pltpu.make_async_copy(
                    table_hbm.at[tok_v.at[j]], bufs[b], gsems[b]).wait()
                pltpu.async_copy(
                    bufs[b], acc.at[dest_v.at[j]], ssems[b], add=True)
            for b in range(_NB):
                j = _NB * i + b
                pltpu.make_async_copy(
                    bufs[b], acc.at[dest_v.at[j]], ssems[b]).wait()
                pltpu.async_copy(
                    table_hbm.at[tok_v.at[j + _NB]], bufs[b], gsems[b])
            return carry

        lax.fori_loop(0, _NCHUNK // _NB - 1, body, 0)

        for b in range(_NB):
            j = _NCHUNK - _NB + b
            pltpu.make_async_copy(
                table_hbm.at[tok_v.at[j]], bufs[b], gsems[b]).wait()
            pltpu.async_copy(bufs[b], acc.at[dest_v.at[j]], ssems[b], add=True)
        for b in range(_NB):
            j = _NCHUNK - _NB + b
            pltpu.make_async_copy(
                bufs[b], acc.at[dest_v.at[j]], ssems[b]).wait()

        base = w * _R
        sbase = s * 3 * _R
        pltpu.sync_copy(acc.at[pl.ds(sbase, _R)], out_q.at[pl.ds(base, _R)])
        pltpu.sync_copy(acc.at[pl.ds(sbase + _R, _R)], out_g.at[pl.ds(base, _R)])
        pltpu.sync_copy(acc.at[pl.ds(sbase + 2 * _R, _R)], out_b.at[pl.ds(base, _R)])

    return pool


_pool_kernel = _make_pool_kernel()


_V = 1000000
_MAIN = 999936


def _eye128():
    return (jax.lax.broadcasted_iota(jnp.int32, (128, 128), 0) ==
            jax.lax.broadcasted_iota(jnp.int32, (128, 128), 1)).astype(jnp.float32)


def _make_transpose_body(groups, n_aliased=0):
    def body(*refs):
        a_ref, out_ref = refs[n_aliased], refs[n_aliased + 1]
        eye = _eye128()
        for g in range(groups):
            ab = jnp.concatenate(
                [a_ref[:, 256 * g: 256 * g + 128],
                 a_ref[:, 256 * g + 128: 256 * g + 256]], axis=0)
            hi = jax.lax.bitcast_convert_type(
                jax.lax.bitcast_convert_type(ab, jnp.int32) & (-65536),
                jnp.float32)
            lo = ab - hi
            dims = (((0,), (0,)), ((), ()))
            out_ref[128 * g: 128 * (g + 1), :] = (
                jax.lax.dot_general(hi, eye, dims,
                                    preferred_element_type=jnp.float32)
                + jax.lax.dot_general(lo, eye, dims,
                                      preferred_element_type=jnp.float32)
            )
    return body


def _tail_body(_, a_ref, out_ref):
    t = a_ref[...].T
    out_ref[...] = jnp.concatenate([t[0:32], t[32:64]], axis=1)


def _linearize_table(embeddings):
    emb_t = embeddings.T
    f32 = jnp.float32
    lin = pl.pallas_call(
        _make_transpose_body(32),
        grid=(_V // 8192,),
        in_specs=[pl.BlockSpec((_D, 8192), lambda i: (0, i))],
        out_specs=pl.BlockSpec((4096, 128), lambda i: (i, 0)),
        out_shape=jax.ShapeDtypeStruct((_V // 2, 128), f32),
    )(emb_t)
    lin = pl.pallas_call(
        _make_transpose_body(2, n_aliased=1),
        grid=(1,),
        in_specs=[
            pl.BlockSpec(memory_space=pl.ANY),
            pl.BlockSpec((_D, 512), lambda i: (0, 1952)),
        ],
        out_specs=pl.BlockSpec((256, 128), lambda i: (1952, 0)),
        out_shape=jax.ShapeDtypeStruct((_V // 2, 128), f32),
        input_output_aliases={0: 0},
    )(lin, emb_t)
    tail = jax.lax.slice(emb_t, (0, _MAIN), (_D, _V))
    lin = pl.pallas_call(
        _tail_body,
        grid=(1,),
        in_specs=[
            pl.BlockSpec(memory_space=pl.ANY),
            pl.BlockSpec((_D, 64), lambda i: (0, 0)),
        ],
        out_specs=pl.BlockSpec((32, 128), lambda i: (_MAIN // 64, 0)),
        out_shape=jax.ShapeDtypeStruct((_V // 2, 128), f32),
        input_output_aliases={0: 0},
    )(lin, tail)
    return lin.reshape(_V, _D)


def _sigma(r):
    p = r - _MAIN
    return jnp.where(
        r < _MAIN,
        2 * (128 * (r // 256) + r % 128) + (r % 256) // 128,
        _MAIN + 2 * (p % 32) + p // 32,
    )


def _sim_body(q_ref, g_ref, b_ref, li_ref, pred_ref, loss_ref):
    q = q_ref[...]
    g = g_ref[...]
    b = b_ref[...]
    qg = jnp.sum(q * g, axis=1, keepdims=True)
    qb = jnp.sum(q * b, axis=1, keepdims=True)
    qq = jnp.sum(q * q, axis=1, keepdims=True)
    gg = jnp.sum(g * g, axis=1, keepdims=True)
    bb = jnp.sum(b * b, axis=1, keepdims=True)
    rq = lax.rsqrt(qq)
    sim_g = qg * rq * lax.rsqrt(gg)
    sim_b = qb * rq * lax.rsqrt(bb)
    li = jnp.maximum(0.0, _MARGIN - sim_g + sim_b)
    li_ref[...] = li
    pred_ref[...] = sim_g
    loss_ref[...] = jnp.sum(li, axis=0, keepdims=True) * (1.0 / _B)


def kernel(input_question, input_answer_good, input_answer_bad, embeddings):
    q = input_question.reshape(_NW, _R * _QL)
    g = input_answer_good.reshape(_NW, _R * _AL)
    b = input_answer_bad.reshape(_NW, _R * _AL)
    tok = jnp.concatenate([q, g, b], axis=1).reshape(_NW, _NCHUNK, _CH)
    tok = _sigma(tok)
    dest = jnp.asarray(_dest_per_subcore)
    zeros = jnp.zeros((3 * _R, _D), jnp.float32)

    table_lin = _linearize_table(embeddings)
    sum_q, sum_g, sum_b = _pool_kernel(tok, dest, zeros, table_lin)

    f32 = jnp.float32
    li, pred, loss = pl.pallas_call(
        _sim_body,
        out_shape=[
            jax.ShapeDtypeStruct((_B, 1), f32),
            jax.ShapeDtypeStruct((_B, 1), f32),
            jax.ShapeDtypeStruct((1, 1), f32),
        ],
    )(sum_q, sum_g, sum_b)

    return (loss[0, 0], li[:, 0], pred[:, 0])


# speedup vs baseline: 2.0695x; 1.1248x over previous
import functools

import jax
import jax.numpy as jnp
import numpy as np
from jax import lax
from jax.experimental import pallas as pl
from jax.experimental.pallas import tpu as pltpu
from jax.experimental.pallas import tpu_sc as plsc

_MARGIN = 0.2
_B, _QL, _AL, _D = 4096, 20, 50, 64
_NC, _NS = 2, 16
_NW = _NC * _NS
_R = _B // _NW
_CH = 128
_TOK = _R * (_QL + 2 * _AL)
_NCHUNK = _TOK // _CH
_NB = 8

_dest = np.concatenate([
    np.arange(_R * _QL, dtype=np.int32) // _QL,
    _R + np.arange(_R * _AL, dtype=np.int32) // _AL,
    2 * _R + np.arange(_R * _AL, dtype=np.int32) // _AL,
]).reshape(_NCHUNK, _CH)
_dest_per_subcore = (
    np.arange(_NS, dtype=np.int32)[:, None, None] * (3 * _R) + _dest[None]
)


def _make_pool_kernel():
    mesh = plsc.VectorSubcoreMesh(core_axis_name="c", subcore_axis_name="s")
    f32 = jnp.float32

    @functools.partial(
        pl.kernel,
        out_type=[jax.ShapeDtypeStruct((_B, _D), f32)] * 3,
        mesh=mesh,
        scratch_types=[
            pltpu.VMEM((_NCHUNK, _CH), jnp.int32),
            pltpu.VMEM((_NCHUNK, _CH), jnp.int32),
            [pltpu.VMEM((_CH, _D), f32)] * _NB,
            pltpu.VMEM_SHARED((_NS * 3 * _R, _D), f32),
            [pltpu.SemaphoreType.DMA] * _NB,
            [pltpu.SemaphoreType.DMA] * _NB,
        ],
        compiler_params=pltpu.CompilerParams(use_tc_tiling_on_sc=False),
    )
    def pool(tok_hbm, dest_hbm, zeros_hbm, table_hbm, out_q, out_g, out_b,
             tok_v, dest_v, bufs, acc, gsems, ssems):
        c = lax.axis_index("c")
        s = lax.axis_index("s")
        w = s * _NC + c

        pltpu.sync_copy(tok_hbm.at[w], tok_v)
        pltpu.sync_copy(dest_hbm.at[s], dest_v)
        pltpu.sync_copy(zeros_hbm, acc.at[pl.ds(s * 3 * _R, 3 * _R)])

        for b in range(_NB):
            pltpu.async_copy(table_hbm.at[tok_v.at[b]], bufs[b], gsems[b])

        def body(i, carry):
            for b in range(_NB):
                j = _NB * i + b
                pltpu.make_async_copy(
                    table_hbm.at[tok_v.at[j]], bufs[b], gsems[b]).wait()
                pltpu.async_copy(
                    bufs[b], acc.at[dest_v.at[j]], ssems[b], add=True)
            for b in range(_NB):
                j = _NB * i + b
                pltpu.make_async_copy(
                    bufs[b], acc.at[dest_v.at[j]], ssems[b]).wait()
                pltpu.async_copy(
                    table_hbm.at[tok_v.at[j + _NB]], bufs[b], gsems[b])
            return carry

        lax.fori_loop(0, _NCHUNK // _NB - 1, body, 0)

        for b in range(_NB):
            j = _NCHUNK - _NB + b
            pltpu.make_async_copy(
                table_hbm.at[tok_v.at[j]], bufs[b], gsems[b]).wait()
            pltpu.async_copy(bufs[b], acc.at[dest_v.at[j]], ssems[b], add=True)
        for b in range(_NB):
            j = _NCHUNK - _NB + b
            pltpu.make_async_copy(
                bufs[b], acc.at[dest_v.at[j]], ssems[b]).wait()

        base = w * _R
        sbase = s * 3 * _R
        pltpu.sync_copy(acc.at[pl.ds(sbase, _R)], out_q.at[pl.ds(base, _R)])
        pltpu.sync_copy(acc.at[pl.ds(sbase + _R, _R)], out_g.at[pl.ds(base, _R)])
        pltpu.sync_copy(acc.at[pl.ds(sbase + 2 * _R, _R)], out_b.at[pl.ds(base, _R)])

    return pool


_pool_kernel = _make_pool_kernel()


_V = 1000000
_MAIN = 999936


def _eye128():
    return (jax.lax.broadcasted_iota(jnp.int32, (128, 128), 0) ==
            jax.lax.broadcasted_iota(jnp.int32, (128, 128), 1)).astype(jnp.float32)


def _make_transpose_body(groups, n_aliased=0):
    def body(*refs):
        a_ref, out_ref = refs[n_aliased], refs[n_aliased + 1]
        eye = _eye128()
        for g in range(groups):
            ab = jnp.concatenate(
                [a_ref[:, 256 * g: 256 * g + 128],
                 a_ref[:, 256 * g + 128: 256 * g + 256]], axis=0)
            hi = jax.lax.bitcast_convert_type(
                jax.lax.bitcast_convert_type(ab, jnp.int32) & (-65536),
                jnp.float32)
            lo = ab - hi
            dims = (((0,), (0,)), ((), ()))
            out_ref[128 * g: 128 * (g + 1), :] = (
                jax.lax.dot_general(hi, eye, dims,
                                    preferred_element_type=jnp.float32)
                + jax.lax.dot_general(lo, eye, dims,
                                      preferred_element_type=jnp.float32)
            )
    return body


def _tail_body(_, a_ref, out_ref):
    t = a_ref[...].T
    out_ref[...] = jnp.concatenate([t[0:32], t[32:64]], axis=1)


def _linearize_table(embeddings):
    emb_t = embeddings.T
    f32 = jnp.float32
    lin = pl.pallas_call(
        _make_transpose_body(64),
        grid=(_V // 16384,),
        in_specs=[pl.BlockSpec((_D, 16384), lambda i: (0, i))],
        out_specs=pl.BlockSpec((8192, 128), lambda i: (i, 0)),
        out_shape=jax.ShapeDtypeStruct((_V // 2, 128), f32),
    )(emb_t)
    lin = pl.pallas_call(
        _make_transpose_body(2, n_aliased=1),
        grid=(1,),
        in_specs=[
            pl.BlockSpec(memory_space=pl.ANY),
            pl.BlockSpec((_D, 512), lambda i: (0, 1952)),
        ],
        out_specs=pl.BlockSpec((256, 128), lambda i: (1952, 0)),
        out_shape=jax.ShapeDtypeStruct((_V // 2, 128), f32),
        input_output_aliases={0: 0},
    )(lin, emb_t)
    tail = jax.lax.slice(emb_t, (0, _MAIN), (_D, _V))
    lin = pl.pallas_call(
        _tail_body,
        grid=(1,),
        in_specs=[
            pl.BlockSpec(memory_space=pl.ANY),
            pl.BlockSpec((_D, 64), lambda i: (0, 0)),
        ],
        out_specs=pl.BlockSpec((32, 128), lambda i: (_MAIN // 64, 0)),
        out_shape=jax.ShapeDtypeStruct((_V // 2, 128), f32),
        input_output_aliases={0: 0},
    )(lin, tail)
    return lin.reshape(_V, _D)


def _sigma(r):
    p = r - _MAIN
    return jnp.where(
        r < _MAIN,
        2 * (128 * (r // 256) + r % 128) + (r % 256) // 128,
        _MAIN + 2 * (p % 32) + p // 32,
    )


def _sim_body(q_ref, g_ref, b_ref, li_ref, pred_ref, loss_ref):
    q = q_ref[...]
    g = g_ref[...]
    b = b_ref[...]
    qg = jnp.sum(q * g, axis=1, keepdims=True)
    qb = jnp.sum(q * b, axis=1, keepdims=True)
    qq = jnp.sum(q * q, axis=1, keepdims=True)
    gg = jnp.sum(g * g, axis=1, keepdims=True)
    bb = jnp.sum(b * b, axis=1, keepdims=True)
    rq = lax.rsqrt(qq)
    sim_g = qg * rq * lax.rsqrt(gg)
    sim_b = qb * rq * lax.rsqrt(bb)
    li = jnp.maximum(0.0, _MARGIN - sim_g + sim_b)
    li_ref[...] = li
    pred_ref[...] = sim_g
    loss_ref[...] = jnp.sum(li, axis=0, keepdims=True) * (1.0 / _B)


def kernel(input_question, input_answer_good, input_answer_bad, embeddings):
    q = input_question.reshape(_NW, _R * _QL)
    g = input_answer_good.reshape(_NW, _R * _AL)
    b = input_answer_bad.reshape(_NW, _R * _AL)
    tok = jnp.concatenate([q, g, b], axis=1).reshape(_NW, _NCHUNK, _CH)
    tok = _sigma(tok)
    dest = jnp.asarray(_dest_per_subcore)
    zeros = jnp.zeros((3 * _R, _D), jnp.float32)

    table_lin = _linearize_table(embeddings)
    sum_q, sum_g, sum_b = _pool_kernel(tok, dest, zeros, table_lin)

    f32 = jnp.float32
    li, pred, loss = pl.pallas_call(
        _sim_body,
        out_shape=[
            jax.ShapeDtypeStruct((_B, 1), f32),
            jax.ShapeDtypeStruct((_B, 1), f32),
            jax.ShapeDtypeStruct((1, 1), f32),
        ],
    )(sum_q, sum_g, sum_b)

    return (loss[0, 0], li[:, 0], pred[:, 0])


# speedup vs baseline: 2.0781x; 1.0041x over previous
import functools

import jax
import jax.numpy as jnp
import numpy as np
from jax import lax
from jax.experimental import pallas as pl
from jax.experimental.pallas import tpu as pltpu
from jax.experimental.pallas import tpu_sc as plsc

_MARGIN = 0.2
_B, _QL, _AL, _D = 4096, 20, 50, 64
_NC, _NS = 2, 16
_NW = _NC * _NS
_R = _B // _NW
_CH = 128
_TOK = _R * (_QL + 2 * _AL)
_NCHUNK = _TOK // _CH
_NB = 8

_dest = np.concatenate([
    np.arange(_R * _QL, dtype=np.int32) // _QL,
    _R + np.arange(_R * _AL, dtype=np.int32) // _AL,
    2 * _R + np.arange(_R * _AL, dtype=np.int32) // _AL,
]).reshape(_NCHUNK, _CH)
_dest_per_subcore = (
    np.arange(_NS, dtype=np.int32)[:, None, None] * (3 * _R) + _dest[None]
)


def _make_pool_kernel():
    mesh = plsc.VectorSubcoreMesh(core_axis_name="c", subcore_axis_name="s")
    f32 = jnp.float32

    @functools.partial(
        pl.kernel,
        out_type=[jax.ShapeDtypeStruct((_B, _D), f32)] * 3,
        mesh=mesh,
        scratch_types=[
            pltpu.VMEM((_NCHUNK, _CH), jnp.int32),
            pltpu.VMEM((_NCHUNK, _CH), jnp.int32),
            [pltpu.VMEM((_CH, _D), f32)] * _NB,
            pltpu.VMEM_SHARED((_NS * 3 * _R, _D), f32),
            [pltpu.SemaphoreType.DMA] * _NB,
            [pltpu.SemaphoreType.DMA] * _NB,
        ],
        compiler_params=pltpu.CompilerParams(use_tc_tiling_on_sc=False),
    )
    def pool(tok_hbm, dest_hbm, zeros_hbm, table_hbm, out_q, out_g, out_b,
             tok_v, dest_v, bufs, acc, gsems, ssems):
        c = lax.axis_index("c")
        s = lax.axis_index("s")
        w = s * _NC + c

        pltpu.sync_copy(tok_hbm.at[w], tok_v)
        pltpu.sync_copy(dest_hbm.at[s], dest_v)
        pltpu.sync_copy(zeros_hbm.at[pl.ds(s * 3 * _R, 3 * _R)],
                        acc.at[pl.ds(s * 3 * _R, 3 * _R)])

        for b in range(_NB):
            pltpu.async_copy(table_hbm.at[tok_v.at[b]], bufs[b], gsems[b])

        def body(i, carry):
            for b in range(_NB):
                j = _NB * i + b
                pltpu.make_async_copy(
                    table_hbm.at[tok_v.at[j]], bufs[b], gsems[b]).wait()
                pltpu.async_copy(
                    bufs[b], acc.at[dest_v.at[j]], ssems[b], add=True)
            for b in range(_NB):
                j = _NB * i + b
                pltpu.make_async_copy(
                    bufs[b], acc.at[dest_v.at[j]], ssems[b]).wait()
                pltpu.async_copy(
                    table_hbm.at[tok_v.at[j + _NB]], bufs[b], gsems[b])
            return carry

        lax.fori_loop(0, _NCHUNK // _NB - 1, body, 0)

        for b in range(_NB):
            j = _NCHUNK - _NB + b
            pltpu.make_async_copy(
                table_hbm.at[tok_v.at[j]], bufs[b], gsems[b]).wait()
            pltpu.async_copy(bufs[b], acc.at[dest_v.at[j]], ssems[b], add=True)
        for b in range(_NB):
            j = _NCHUNK - _NB + b
            pltpu.make_async_copy(
                bufs[b], acc.at[dest_v.at[j]], ssems[b]).wait()

        base = w * _R
        sbase = s * 3 * _R
        pltpu.sync_copy(acc.at[pl.ds(sbase, _R)], out_q.at[pl.ds(base, _R)])
        pltpu.sync_copy(acc.at[pl.ds(sbase + _R, _R)], out_g.at[pl.ds(base, _R)])
        pltpu.sync_copy(acc.at[pl.ds(sbase + 2 * _R, _R)], out_b.at[pl.ds(base, _R)])

    return pool


_pool_kernel = _make_pool_kernel()


_V = 1000000
_MAIN = 999936


def _eye128():
    return (jax.lax.broadcasted_iota(jnp.int32, (128, 128), 0) ==
            jax.lax.broadcasted_iota(jnp.int32, (128, 128), 1)).astype(jnp.float32)


def _make_transpose_body(groups, n_aliased=0):
    def body(*refs):
        a_ref, out_ref = refs[n_aliased], refs[n_aliased + 1]
        eye = _eye128()
        for g in range(groups):
            ab = jnp.concatenate(
                [a_ref[:, 256 * g: 256 * g + 128],
                 a_ref[:, 256 * g + 128: 256 * g + 256]], axis=0)
            hi = jax.lax.bitcast_convert_type(
                jax.lax.bitcast_convert_type(ab, jnp.int32) & (-65536),
                jnp.float32)
            lo = ab - hi
            dims = (((0,), (0,)), ((), ()))
            out_ref[128 * g: 128 * (g + 1), :] = (
                jax.lax.dot_general(hi, eye, dims,
                                    preferred_element_type=jnp.float32)
                + jax.lax.dot_general(lo, eye, dims,
                                      preferred_element_type=jnp.float32)
            )
    return body


def _tail_body(_, a_ref, out_ref):
    t = a_ref[...].T
    out_ref[...] = jnp.concatenate([t[0:32], t[32:64]], axis=1)


def _linearize_table(embeddings):
    emb_t = embeddings.T
    f32 = jnp.float32
    lin = pl.pallas_call(
        _make_transpose_body(64),
        grid=(_V // 16384,),
        in_specs=[pl.BlockSpec((_D, 16384), lambda i: (0, i))],
        out_specs=pl.BlockSpec((8192, 128), lambda i: (i, 0)),
        out_shape=jax.ShapeDtypeStruct((_V // 2, 128), f32),
    )(emb_t)
    lin = pl.pallas_call(
        _make_transpose_body(2, n_aliased=1),
        grid=(1,),
        in_specs=[
            pl.BlockSpec(memory_space=pl.ANY),
            pl.BlockSpec((_D, 512), lambda i: (0, 1952)),
        ],
        out_specs=pl.BlockSpec((256, 128), lambda i: (1952, 0)),
        out_shape=jax.ShapeDtypeStruct((_V // 2, 128), f32),
        input_output_aliases={0: 0},
    )(lin, emb_t)
    tail = jax.lax.slice(emb_t, (0, _MAIN), (_D, _V))
    lin = pl.pallas_call(
        _tail_body,
        grid=(1,),
        in_specs=[
            pl.BlockSpec(memory_space=pl.ANY),
            pl.BlockSpec((_D, 64), lambda i: (0, 0)),
        ],
        out_specs=pl.BlockSpec((32, 128), lambda i: (_MAIN // 64, 0)),
        out_shape=jax.ShapeDtypeStruct((_V // 2, 128), f32),
        input_output_aliases={0: 0},
    )(lin, tail)
    return lin.reshape(_V, _D)


def _sigma(r):
    p = r - _MAIN
    return jnp.where(
        r < _MAIN,
        2 * (128 * (r // 256) + r % 128) + (r % 256) // 128,
        _MAIN + 2 * (p % 32) + p // 32,
    )


def _sim_body(q_ref, g_ref, b_ref, li_ref, pred_ref, loss_ref):
    q = q_ref[...]
    g = g_ref[...]
    b = b_ref[...]
    qg = jnp.sum(q * g, axis=1, keepdims=True)
    qb = jnp.sum(q * b, axis=1, keepdims=True)
    qq = jnp.sum(q * q, axis=1, keepdims=True)
    gg = jnp.sum(g * g, axis=1, keepdims=True)
    bb = jnp.sum(b * b, axis=1, keepdims=True)
    rq = lax.rsqrt(qq)
    sim_g = qg * rq * lax.rsqrt(gg)
    sim_b = qb * rq * lax.rsqrt(bb)
    li = jnp.maximum(0.0, _MARGIN - sim_g + sim_b)
    li_ref[...] = li
    pred_ref[...] = sim_g
    loss_ref[...] = jnp.sum(li, axis=0, keepdims=True) * (1.0 / _B)


def kernel(input_question, input_answer_good, input_answer_bad, embeddings):
    q = input_question.reshape(_NW, _R * _QL)
    g = input_answer_good.reshape(_NW, _R * _AL)
    b = input_answer_bad.reshape(_NW, _R * _AL)
    tok = jnp.concatenate([q, g, b], axis=1).reshape(_NW, _NCHUNK, _CH)
    tok = _sigma(tok)
    dest = jnp.asarray(_dest_per_subcore)
    zeros = jnp.zeros((_NS * 3 * _R, _D), jnp.float32)

    table_lin = _linearize_table(embeddings)
    sum_q, sum_g, sum_b = _pool_kernel(tok, dest, zeros, table_lin)

    f32 = jnp.float32
    li, pred, loss = pl.pallas_call(
        _sim_body,
        out_shape=[
            jax.ShapeDtypeStruct((_B, 1), f32),
            jax.ShapeDtypeStruct((_B, 1), f32),
            jax.ShapeDtypeStruct((1, 1), f32),
        ],
    )(sum_q, sum_g, sum_b)

    return (loss[0, 0], li[:, 0], pred[:, 0])
